# baseline (speedup 1.0000x reference)
import functools
import math

import jax
import jax.numpy as jnp
from jax import lax
from jax.experimental import pallas as pl
from jax.experimental.pallas import tpu as pltpu
from jax.experimental.pallas import tpu_sc as plsc

B, L, DIN, D, K, TOPK = 8, 1024, 768, 64, 8192, 5
N = B * L
TB = 512
TB3 = 512
KC = 2048
NBLK = N // TB
NBLK3 = N // TB3


def _xq_argmin_body(img_ref, w1t_ref, b1_ref, g1_ref, be1_ref, embt_ref,
                    esq_ref, xq_ref, idx_ref):
    x = img_ref[...]
    h = jnp.dot(x, w1t_ref[...], preferred_element_type=jnp.float32)
    h = h + b1_ref[...]
    m = jnp.mean(h, axis=1, keepdims=True)
    c = h - m
    v = jnp.mean(c * c, axis=1, keepdims=True)
    xq = c / jnp.sqrt(v + 1e-5) * g1_ref[...] + be1_ref[...]
    xq = jnp.maximum(xq, 0.0)

    fsq = jnp.sum(xq * xq, axis=1, keepdims=True)
    rmin = None
    ridx = None
    for ci in range(K // KC):
        et = embt_ref[:, ci * KC:(ci + 1) * KC]
        esq = esq_ref[:, ci * KC:(ci + 1) * KC]
        mm2 = jnp.dot(xq, et, preferred_element_type=jnp.float32)
        d2 = (fsq + esq) - mm2
        cmin = jnp.min(d2, axis=1, keepdims=True)
        io = lax.broadcasted_iota(jnp.int32, (TB, KC), 1)
        cidx = jnp.min(jnp.where(d2 <= cmin, io, jnp.int32(1 << 30)),
                       axis=1, keepdims=True) + ci * KC
        if ci == 0:
            rmin, ridx = cmin, cidx
        else:
            upd = cmin < rmin
            rmin = jnp.where(upd, cmin, rmin)
            ridx = jnp.where(upd, cidx, ridx)
    xq_ref[...] = xq
    idx_ref[...] = ridx


def _call_k1(img2d, w1t, b1r, g1r, be1r, embt, esqr):
    return pl.pallas_call(
        _xq_argmin_body,
        grid=(NBLK,),
        in_specs=[
            pl.BlockSpec((TB, DIN), lambda i: (i, 0)),
            pl.BlockSpec((DIN, D), lambda i: (0, 0)),
            pl.BlockSpec((1, D), lambda i: (0, 0)),
            pl.BlockSpec((1, D), lambda i: (0, 0)),
            pl.BlockSpec((1, D), lambda i: (0, 0)),
            pl.BlockSpec((D, K), lambda i: (0, 0)),
            pl.BlockSpec((1, K), lambda i: (0, 0)),
        ],
        out_specs=[
            pl.BlockSpec((TB, D), lambda i: (i, 0)),
            pl.BlockSpec((TB, 1), lambda i: (i, 0)),
        ],
        out_shape=[
            jax.ShapeDtypeStruct((N, D), jnp.float32),
            jax.ShapeDtypeStruct((N, 1), jnp.int32),
        ],
    )(img2d, w1t, b1r, g1r, be1r, embt, esqr)


def _sampling_body(mp_ref, xq_ref, embt_ref, esq_ref, idxc_ref, idx2_ref,
                   g_ref, neg_ref):
    rows = []
    grows = []
    cbs = []
    for b in range(B):
        mpos = mp_ref[b, 0]
        t = b * L + mpos
        rows.append(xq_ref[pl.ds(t, 1), :])
        grows.append(g_ref[pl.ds(t, 1), :])
        cbs.append(idxc_ref[pl.ds(t, 1), :])
    rows = jnp.concatenate(rows, axis=0)
    grows = jnp.concatenate(grows, axis=0)
    cb = jnp.concatenate(cbs, axis=0)

    et = embt_ref[...]
    esq = esq_ref[...]
    fsq = jnp.sum(rows * rows, axis=1, keepdims=True)
    mm2 = jnp.dot(rows, et, preferred_element_type=jnp.float32)
    d2 = (fsq + esq) - mm2
    nl = -d2
    mx = jnp.max(nl, axis=1, keepdims=True)
    p = jnp.exp(nl - mx)
    probs = p / jnp.sum(p, axis=1, keepdims=True)

    io = lax.broadcasted_iota(jnp.int32, (B, K), 1)
    pw = probs
    vals = []
    idxs = []
    for _ in range(TOPK):
        vmax = jnp.max(pw, axis=1, keepdims=True)
        ik = jnp.min(jnp.where(pw >= vmax, io, jnp.int32(1 << 30)),
                     axis=1, keepdims=True)
        vals.append(vmax)
        idxs.append(ik)
        pw = jnp.where(io == ik, -1.0, pw)

    best_l = jnp.log(vals[0] + 1e-12) + grows[:, 0:1]
    best_k = jnp.zeros((B, 1), jnp.int32)
    for k in range(1, TOPK):
        lk = jnp.log(vals[k] + 1e-12) + grows[:, k:k + 1]
        upd = lk > best_l
        best_l = jnp.where(upd, lk, best_l)
        best_k = jnp.where(upd, jnp.int32(k), best_k)
    ng = jnp.zeros((B, 1), jnp.int32)
    for k in range(TOPK):
        ng = jnp.where(best_k == k, idxs[k], ng)

    ind2d = idx2_ref[...]
    neg_ref[...] = jnp.where(ind2d == cb, ng, ind2d)


def _call_k2(mask_pos, xq, embt, esqr, idxcol, idx2d, gpad):
    return pl.pallas_call(
        _sampling_body,
        in_specs=[
            pl.BlockSpec(memory_space=pltpu.SMEM),
            pl.BlockSpec((N, D), lambda: (0, 0)),
            pl.BlockSpec((D, K), lambda: (0, 0)),
            pl.BlockSpec((1, K), lambda: (0, 0)),
            pl.BlockSpec((N, 1), lambda: (0, 0)),
            pl.BlockSpec((B, L), lambda: (0, 0)),
            pl.BlockSpec((N, 8), lambda: (0, 0)),
        ],
        out_specs=pl.BlockSpec((B, L), lambda: (0, 0)),
        out_shape=jax.ShapeDtypeStruct((B, L), jnp.int32),
    )(mask_pos, xq, embt, esqr, idxcol, idx2d, gpad)


def _table_body(emb_ref, wpt_ref, out_ref):
    e = emb_ref[...]
    out_ref[:, :D] = e.astype(jnp.bfloat16).astype(jnp.float32)
    out_ref[:, D:] = jnp.dot(e, wpt_ref[...],
                             preferred_element_type=jnp.float32)


def _call_k0(embed, wpt):
    return pl.pallas_call(
        _table_body,
        in_specs=[
            pl.BlockSpec((K, D), lambda: (0, 0)),
            pl.BlockSpec((D, D), lambda: (0, 0)),
        ],
        out_specs=pl.BlockSpec((K, 2 * D), lambda: (0, 0)),
        out_shape=jax.ShapeDtypeStruct((K, 2 * D), jnp.float32),
    )(embed, wpt)


_SC_ROWS = 256
_SC_CHUNK = 256


def _sc_gather_two(table, idx_a, idx_b):
    info = plsc.get_sparse_core_info()
    nw = info.num_cores * info.num_subcores
    mesh = plsc.VectorSubcoreMesh(core_axis_name="c", subcore_axis_name="s")

    nj = _SC_ROWS // _SC_CHUNK

    @functools.partial(
        pl.kernel,
        mesh=mesh,
        out_type=[
            jax.ShapeDtypeStruct((N, 2 * D), jnp.float32),
            jax.ShapeDtypeStruct((N, 2 * D), jnp.float32),
        ],
        scratch_types=(
            [pltpu.VMEM((_SC_CHUNK,), jnp.int32) for _ in range(2 * nj)]
            + [pltpu.VMEM((_SC_CHUNK, 2 * D), jnp.float32) for _ in range(2 * nj)]
            + [pltpu.SemaphoreType.DMA] * 3 * 2 * (_SC_ROWS // _SC_CHUNK)
        ),
    )
    def gather_k(table_hbm, ia_hbm, ib_hbm, out_a, out_b, *scratch):
        idx_vs = scratch[:2 * nj]
        row_vs = scratch[2 * nj:4 * nj]
        sems = scratch[4 * nj:]
        sem_i = sems[0:2 * nj]
        sem_g = sems[2 * nj:4 * nj]
        sem_s = sems[4 * nj:6 * nj]
        wid = lax.axis_index("s") * info.num_cores + lax.axis_index("c")
        plan = []
        k = 0
        for src, dst in ((ia_hbm, out_a), (ib_hbm, out_b)):
            for j in range(nj):
                base = wid * _SC_ROWS + j * _SC_CHUNK
                plan.append((src, dst, base, idx_vs[k], row_vs[k]))
                k += 1
        d = [pltpu.async_copy(src.at[pl.ds(base, _SC_CHUNK)], iv, sem_i[t])
             for t, (src, _, base, iv, _) in enumerate(plan)]
        g = []
        for t, (_, _, _, iv, rv) in enumerate(plan):
            d[t].wait()
            g.append(pltpu.async_copy(table_hbm.at[iv], rv, sem_g[t]))
        st = []
        for t, (_, dst, base, _, rv) in enumerate(plan):
            g[t].wait()
            st.append(pltpu.async_copy(rv, dst.at[pl.ds(base, _SC_CHUNK)],
                                       sem_s[t]))
        for x in st:
            x.wait()

    assert N % (8 * nw) == 0
    return gather_k(table, idx_a, idx_b)


def _assemble_body(q_ref, n_ref, xq_ref, pos_ref, wgt_ref, bp_ref, bg_ref,
                   lgg_ref, lgb_ref, lng_ref, lnb_ref, o1_ref, o2_ref):
    x = xq_ref[...]
    e = q_ref[:, D:] + bp_ref[...]
    nq = n_ref[:, :D]
    pos = pos_ref[...]

    tmp = jnp.concatenate([e, x], axis=1)
    sg = jnp.dot(tmp, wgt_ref[...],
                 preferred_element_type=jnp.float32) + bg_ref[...]
    s0 = sg[:, 0:1]
    s1 = sg[:, 1:2]
    mu = (s0 + s1) * 0.5
    d0 = s0 - mu
    d1 = s1 - mu
    var = (d0 * d0 + d1 * d1) * 0.5
    rs = jnp.sqrt(var + 1e-5)
    r0 = jnp.maximum(d0 / rs * lgg_ref[:, 0:1] + lgb_ref[:, 0:1], 0.0)
    r1 = jnp.maximum(d1 / rs * lgg_ref[:, 1:2] + lgb_ref[:, 1:2], 0.0)
    smx = jnp.maximum(r0, r1)
    e0 = jnp.exp(r0 - smx)
    e1 = jnp.exp(r1 - smx)
    den = e0 + e1
    es = e0 / den
    iss = e1 / den

    def final_ln(o):
        m = jnp.mean(o, axis=1, keepdims=True)
        c = o - m
        v = jnp.mean(c * c, axis=1, keepdims=True)
        return c / jnp.sqrt(v + 1e-5) * lng_ref[...] + lnb_ref[...]

    o1_ref[...] = final_ln(e * es + x * iss + pos)
    o2_ref[...] = final_ln(nq * es + x * iss + pos)


def _call_k3(qrows, nrows, xq, pos2d, wgt, bpr, bgr, lggr, lgbr, lngr, lnbr):
    tok = lambda i: (i, 0)
    cst = lambda i: (0, 0)
    return pl.pallas_call(
        _assemble_body,
        grid=(NBLK3,),
        in_specs=[
            pl.BlockSpec((TB3, 2 * D), tok),
            pl.BlockSpec((TB3, 2 * D), tok),
            pl.BlockSpec((TB3, D), tok),
            pl.BlockSpec((TB3, D), tok),
            pl.BlockSpec((2 * D, 2), cst),
            pl.BlockSpec((1, D), cst),
            pl.BlockSpec((1, 2), cst),
            pl.BlockSpec((1, 2), cst),
            pl.BlockSpec((1, 2), cst),
            pl.BlockSpec((1, D), cst),
            pl.BlockSpec((1, D), cst),
        ],
        out_specs=[
            pl.BlockSpec((TB3, D), tok),
            pl.BlockSpec((TB3, D), tok),
        ],
        out_shape=[
            jax.ShapeDtypeStruct((N, D), jnp.float32),
            jax.ShapeDtypeStruct((N, D), jnp.float32),
        ],
    )(qrows, nrows, xq, pos2d, wgt, bpr, bgr, lggr, lgbr, lngr, lnbr)


def _pos_encoding(b, l, d):
    h = w = int(math.sqrt(l))
    mask = jnp.ones((b, h, w), dtype=jnp.float32)
    y_embed = jnp.cumsum(mask, axis=1)
    x_embed = jnp.cumsum(mask, axis=2)
    eps = 1e-6
    y_embed = y_embed / (y_embed[:, -1:, :] + eps) * 2 * math.pi
    x_embed = x_embed / (x_embed[:, :, -1:] + eps) * 2 * math.pi
    pfd = d // 2
    dim_t = jnp.arange(pfd, dtype=jnp.float32)
    dim_t = 10000.0 ** (2.0 * jnp.floor(dim_t / 2.0) / pfd)
    pos_x = x_embed[:, :, :, None] / dim_t
    pos_y = y_embed[:, :, :, None] / dim_t
    pos_x = jnp.stack((jnp.sin(pos_x[:, :, :, 0::2]),
                       jnp.cos(pos_x[:, :, :, 1::2])), axis=4).reshape(b, h, w, pfd)
    pos_y = jnp.stack((jnp.sin(pos_y[:, :, :, 0::2]),
                       jnp.cos(pos_y[:, :, :, 1::2])), axis=4).reshape(b, h, w, pfd)
    pos = jnp.concatenate((pos_y, pos_x), axis=3)
    return pos.reshape(b, h * w, 2 * pfd)


def _np_threefry2x32(k0, k1, x0, x1):
    import numpy as np
    rot1 = (13, 15, 26, 6)
    rot2 = (17, 29, 16, 24)
    ks0, ks1 = np.uint32(k0), np.uint32(k1)
    ks2 = np.uint32(0x1BD11BDA) ^ ks0 ^ ks1
    x0 = (x0 + ks0).astype(np.uint32)
    x1 = (x1 + ks1).astype(np.uint32)
    add_idx = ((ks1, ks2), (ks2, ks0), (ks0, ks1), (ks1, ks2), (ks2, ks0))
    for r in range(5):
        for rot in (rot1 if r % 2 == 0 else rot2):
            x0 = (x0 + x1).astype(np.uint32)
            x1 = ((x1 << np.uint32(rot)) | (x1 >> np.uint32(32 - rot))).astype(np.uint32)
            x1 = x1 ^ x0
        ka, kb = add_idx[r]
        x0 = (x0 + ka).astype(np.uint32)
        x1 = (x1 + kb + np.uint32(r + 1)).astype(np.uint32)
    return x0, x1


def _np_gumbel(seed_pair, shape):
    import numpy as np
    n = int(np.prod(shape))
    o0, o1 = _np_threefry2x32(seed_pair[0], seed_pair[1],
                              np.zeros(n, np.uint32), np.arange(n, dtype=np.uint32))
    bits = o0 ^ o1
    f = ((bits >> np.uint32(9)) | np.uint32(0x3F800000)).view(np.float32)
    u = f - np.float32(1.0)
    tiny = np.float32(np.finfo(np.float32).tiny)
    u = np.maximum(tiny, (u * (np.float32(1.0) - tiny) + tiny).astype(np.float32))
    return (-np.log(-np.log(u))).astype(np.float32).reshape(shape)


def _np_pos_encoding(b, l, d):
    import numpy as np
    h = w = int(math.sqrt(l))
    f32 = np.float32
    y = np.cumsum(np.ones((b, h, w), f32), axis=1, dtype=f32)
    x = np.cumsum(np.ones((b, h, w), f32), axis=2, dtype=f32)
    eps = f32(1e-6)
    two_pi = f32(2 * math.pi)
    y = (y / (y[:, -1:, :] + eps) * two_pi).astype(f32)
    x = (x / (x[:, :, -1:] + eps) * two_pi).astype(f32)
    pfd = d // 2
    dim_t = np.arange(pfd, dtype=f32)
    dim_t = np.power(f32(10000.0), (f32(2.0) * np.floor(dim_t / f32(2.0)) / f32(pfd))).astype(f32)
    pos_x = (x[:, :, :, None] / dim_t).astype(f32)
    pos_y = (y[:, :, :, None] / dim_t).astype(f32)
    pos_x = np.stack((np.sin(pos_x[:, :, :, 0::2]), np.cos(pos_x[:, :, :, 1::2])),
                     axis=4).astype(f32).reshape(b, h, w, pfd)
    pos_y = np.stack((np.sin(pos_y[:, :, :, 0::2]), np.cos(pos_y[:, :, :, 1::2])),
                     axis=4).astype(f32).reshape(b, h, w, pfd)
    return np.concatenate((pos_y, pos_x), axis=3).reshape(b, h * w, 2 * pfd)


def _host_constants():
    import numpy as np
    g = _np_gumbel((0, 42), (N, TOPK))
    gpad = np.concatenate([g, np.zeros((N, 8 - TOPK), np.float32)], axis=1)
    pos = _np_pos_encoding(B, L, D).reshape(N, D)
    return gpad, pos


_GPAD_CONST, _POS_CONST = _host_constants()


def kernel(img, mask_indices, W1, b1, ln1_g, ln1_b, embed, Wp, bp, Wg, bg,
           lng_g, lng_b, ln_g, ln_b):
    img2d = img.reshape(N, DIN)
    w1t = W1.T
    embt = embed.T * 2.0
    b1r = b1.reshape(1, D)
    g1r = ln1_g.reshape(1, D)
    be1r = ln1_b.reshape(1, D)

    esqr = jnp.sum(embed ** 2, axis=1).reshape(1, K)
    xq, idxcol = _call_k1(img2d, w1t, b1r, g1r, be1r, embt, esqr)
    indices = idxcol[:, 0]
    idx2d = indices.reshape(B, L)

    gpad = jnp.asarray(_GPAD_CONST)
    mask_pos = mask_indices.astype(jnp.int32).reshape(B, 1)

    neg2d = _call_k2(mask_pos, xq, embt, esqr, idxcol, idx2d, gpad)

    table = _call_k0(embed, Wp.T)
    qrows, nrows = _sc_gather_two(table, indices, neg2d.reshape(N))

    pos2d = jnp.asarray(_POS_CONST)
    out1, out2 = _call_k3(
        qrows, nrows, xq, pos2d,
        Wg.T, bp.reshape(1, D), bg.reshape(1, 2),
        lng_g.reshape(1, 2), lng_b.reshape(1, 2),
        ln_g.reshape(1, D), ln_b.reshape(1, D),
    )
    return out1.reshape(B, L, D), out2.reshape(B, L, D)


# speedup vs baseline: 1.0045x; 1.0045x over previous
import functools
import math

import jax
import jax.numpy as jnp
from jax import lax
from jax.experimental import pallas as pl
from jax.experimental.pallas import tpu as pltpu
from jax.experimental.pallas import tpu_sc as plsc

B, L, DIN, D, K, TOPK = 8, 1024, 768, 64, 8192, 5
N = B * L
TB = 512
TB3 = 512
KC = 2048
NBLK = N // TB
NBLK3 = N // TB3


def _xq_argmin_body(img_ref, w1t_ref, b1_ref, g1_ref, be1_ref, embt_ref,
                    esq_ref, xq_ref, idx_ref):
    x = img_ref[...]
    h = jnp.dot(x, w1t_ref[...], preferred_element_type=jnp.float32)
    h = h + b1_ref[...]
    m = jnp.mean(h, axis=1, keepdims=True)
    c = h - m
    v = jnp.mean(c * c, axis=1, keepdims=True)
    xq = c / jnp.sqrt(v + 1e-5) * g1_ref[...] + be1_ref[...]
    xq = jnp.maximum(xq, 0.0)

    fsq = jnp.sum(xq * xq, axis=1, keepdims=True)
    rmin = None
    ridx = None
    for ci in range(K // KC):
        et = embt_ref[:, ci * KC:(ci + 1) * KC]
        esq = esq_ref[:, ci * KC:(ci + 1) * KC]
        mm2 = jnp.dot(xq, et, preferred_element_type=jnp.float32)
        d2 = (fsq + esq) - mm2
        cmin = jnp.min(d2, axis=1, keepdims=True)
        io = lax.broadcasted_iota(jnp.int32, (TB, KC), 1)
        cidx = jnp.min(jnp.where(d2 <= cmin, io, jnp.int32(1 << 30)),
                       axis=1, keepdims=True) + ci * KC
        if ci == 0:
            rmin, ridx = cmin, cidx
        else:
            upd = cmin < rmin
            rmin = jnp.where(upd, cmin, rmin)
            ridx = jnp.where(upd, cidx, ridx)
    xq_ref[...] = xq
    ridx_f = ridx.astype(jnp.float32)
    idx_ref[...] = jnp.transpose(ridx_f, (1, 0)).reshape(1, 1, TB).astype(jnp.int32)


def _call_k1(img2d, w1t, b1r, g1r, be1r, embt, esqr):
    return pl.pallas_call(
        _xq_argmin_body,
        grid=(NBLK,),
        in_specs=[
            pl.BlockSpec((TB, DIN), lambda i: (i, 0)),
            pl.BlockSpec((DIN, D), lambda i: (0, 0)),
            pl.BlockSpec((1, D), lambda i: (0, 0)),
            pl.BlockSpec((1, D), lambda i: (0, 0)),
            pl.BlockSpec((1, D), lambda i: (0, 0)),
            pl.BlockSpec((D, K), lambda i: (0, 0)),
            pl.BlockSpec((1, K), lambda i: (0, 0)),
        ],
        out_specs=[
            pl.BlockSpec((TB, D), lambda i: (i, 0)),
            pl.BlockSpec((1, 1, TB), lambda i: (i, 0, 0)),
        ],
        out_shape=[
            jax.ShapeDtypeStruct((N, D), jnp.float32),
            jax.ShapeDtypeStruct((NBLK, 1, TB), jnp.int32),
        ],
    )(img2d, w1t, b1r, g1r, be1r, embt, esqr)


def _sampling_body(mp_ref, xq_ref, embt_ref, esq_ref, idx2_ref,
                   g_ref, neg_ref):
    rows = []
    grows = []
    mposv = []
    for b in range(B):
        mpos = mp_ref[b, 0]
        t = b * L + mpos
        rows.append(xq_ref[pl.ds(t, 1), :])
        grows.append(g_ref[pl.ds(t, 1), :])
        mposv.append(mpos)
    rows = jnp.concatenate(rows, axis=0)
    grows = jnp.concatenate(grows, axis=0)

    ind2d = idx2_ref[...]
    iol = lax.broadcasted_iota(jnp.int32, (B, L), 1)
    mcol = jnp.concatenate([jnp.full((1, L), m, jnp.int32) for m in mposv], 0)
    oh = (iol == mcol).astype(jnp.float32)
    cb = jnp.sum(oh * ind2d.astype(jnp.float32), axis=1,
                 keepdims=True).astype(jnp.int32)

    et = embt_ref[...]
    esq = esq_ref[...]
    fsq = jnp.sum(rows * rows, axis=1, keepdims=True)
    mm2 = jnp.dot(rows, et, preferred_element_type=jnp.float32)
    d2 = (fsq + esq) - mm2
    nl = -d2
    mx = jnp.max(nl, axis=1, keepdims=True)
    p = jnp.exp(nl - mx)
    probs = p / jnp.sum(p, axis=1, keepdims=True)

    io = lax.broadcasted_iota(jnp.int32, (B, K), 1)
    pw = probs
    vals = []
    idxs = []
    for _ in range(TOPK):
        vmax = jnp.max(pw, axis=1, keepdims=True)
        ik = jnp.min(jnp.where(pw >= vmax, io, jnp.int32(1 << 30)),
                     axis=1, keepdims=True)
        vals.append(vmax)
        idxs.append(ik)
        pw = jnp.where(io == ik, -1.0, pw)

    best_l = jnp.log(vals[0] + 1e-12) + grows[:, 0:1]
    best_k = jnp.zeros((B, 1), jnp.int32)
    for k in range(1, TOPK):
        lk = jnp.log(vals[k] + 1e-12) + grows[:, k:k + 1]
        upd = lk > best_l
        best_l = jnp.where(upd, lk, best_l)
        best_k = jnp.where(upd, jnp.int32(k), best_k)
    ng = jnp.zeros((B, 1), jnp.int32)
    for k in range(TOPK):
        ng = jnp.where(best_k == k, idxs[k], ng)

    neg_ref[...] = jnp.where(ind2d == cb, ng, ind2d)


def _call_k2(mask_pos, xq, embt, esqr, idx2d, gpad):
    return pl.pallas_call(
        _sampling_body,
        in_specs=[
            pl.BlockSpec(memory_space=pltpu.SMEM),
            pl.BlockSpec((N, D), lambda: (0, 0)),
            pl.BlockSpec((D, K), lambda: (0, 0)),
            pl.BlockSpec((1, K), lambda: (0, 0)),
            pl.BlockSpec((B, L), lambda: (0, 0)),
            pl.BlockSpec((N, 8), lambda: (0, 0)),
        ],
        out_specs=pl.BlockSpec((B, L), lambda: (0, 0)),
        out_shape=jax.ShapeDtypeStruct((B, L), jnp.int32),
    )(mask_pos, xq, embt, esqr, idx2d, gpad)


def _table_body(emb_ref, wpt_ref, out_ref):
    e = emb_ref[...]
    out_ref[:, :D] = e.astype(jnp.bfloat16).astype(jnp.float32)
    out_ref[:, D:] = jnp.dot(e, wpt_ref[...],
                             preferred_element_type=jnp.float32)


def _call_k0(embed, wpt):
    return pl.pallas_call(
        _table_body,
        in_specs=[
            pl.BlockSpec((K, D), lambda: (0, 0)),
            pl.BlockSpec((D, D), lambda: (0, 0)),
        ],
        out_specs=pl.BlockSpec((K, 2 * D), lambda: (0, 0)),
        out_shape=jax.ShapeDtypeStruct((K, 2 * D), jnp.float32),
    )(embed, wpt)


_SC_ROWS = 256
_SC_CHUNK = 256


def _sc_gather_two(table, idx_a, idx_b):
    info = plsc.get_sparse_core_info()
    nw = info.num_cores * info.num_subcores
    mesh = plsc.VectorSubcoreMesh(core_axis_name="c", subcore_axis_name="s")

    nj = _SC_ROWS // _SC_CHUNK

    @functools.partial(
        pl.kernel,
        mesh=mesh,
        out_type=[
            jax.ShapeDtypeStruct((N, 2 * D), jnp.float32),
            jax.ShapeDtypeStruct((N, 2 * D), jnp.float32),
        ],
        scratch_types=(
            [pltpu.VMEM((_SC_CHUNK,), jnp.int32) for _ in range(2 * nj)]
            + [pltpu.VMEM((_SC_CHUNK, 2 * D), jnp.float32) for _ in range(2 * nj)]
            + [pltpu.SemaphoreType.DMA] * 3 * 2 * (_SC_ROWS // _SC_CHUNK)
        ),
    )
    def gather_k(table_hbm, ia_hbm, ib_hbm, out_a, out_b, *scratch):
        idx_vs = scratch[:2 * nj]
        row_vs = scratch[2 * nj:4 * nj]
        sems = scratch[4 * nj:]
        sem_i = sems[0:2 * nj]
        sem_g = sems[2 * nj:4 * nj]
        sem_s = sems[4 * nj:6 * nj]
        wid = lax.axis_index("s") * info.num_cores + lax.axis_index("c")
        plan = []
        k = 0
        for src, dst in ((ia_hbm, out_a), (ib_hbm, out_b)):
            for j in range(nj):
                base = wid * _SC_ROWS + j * _SC_CHUNK
                plan.append((src, dst, base, idx_vs[k], row_vs[k]))
                k += 1
        d = [pltpu.async_copy(src.at[pl.ds(base, _SC_CHUNK)], iv, sem_i[t])
             for t, (src, _, base, iv, _) in enumerate(plan)]
        g = []
        for t, (_, _, _, iv, rv) in enumerate(plan):
            d[t].wait()
            g.append(pltpu.async_copy(table_hbm.at[iv], rv, sem_g[t]))
        st = []
        for t, (_, dst, base, _, rv) in enumerate(plan):
            g[t].wait()
            st.append(pltpu.async_copy(rv, dst.at[pl.ds(base, _SC_CHUNK)],
                                       sem_s[t]))
        for x in st:
            x.wait()

    assert N % (8 * nw) == 0
    return gather_k(table, idx_a, idx_b)


def _assemble_body(q_ref, n_ref, xq_ref, pos_ref, wgt_ref, bp_ref, bg_ref,
                   lgg_ref, lgb_ref, lng_ref, lnb_ref, o1_ref, o2_ref):
    x = xq_ref[...]
    e = q_ref[:, D:] + bp_ref[...]
    nq = n_ref[:, :D]
    pos = pos_ref[...]

    tmp = jnp.concatenate([e, x], axis=1)
    sg = jnp.dot(tmp, wgt_ref[...],
                 preferred_element_type=jnp.float32) + bg_ref[...]
    s0 = sg[:, 0:1]
    s1 = sg[:, 1:2]
    mu = (s0 + s1) * 0.5
    d0 = s0 - mu
    d1 = s1 - mu
    var = (d0 * d0 + d1 * d1) * 0.5
    rs = jnp.sqrt(var + 1e-5)
    r0 = jnp.maximum(d0 / rs * lgg_ref[:, 0:1] + lgb_ref[:, 0:1], 0.0)
    r1 = jnp.maximum(d1 / rs * lgg_ref[:, 1:2] + lgb_ref[:, 1:2], 0.0)
    smx = jnp.maximum(r0, r1)
    e0 = jnp.exp(r0 - smx)
    e1 = jnp.exp(r1 - smx)
    den = e0 + e1
    es = e0 / den
    iss = e1 / den

    def final_ln(o):
        m = jnp.mean(o, axis=1, keepdims=True)
        c = o - m
        v = jnp.mean(c * c, axis=1, keepdims=True)
        return c / jnp.sqrt(v + 1e-5) * lng_ref[...] + lnb_ref[...]

    o1_ref[...] = final_ln(e * es + x * iss + pos)
    o2_ref[...] = final_ln(nq * es + x * iss + pos)


def _call_k3(qrows, nrows, xq, pos2d, wgt, bpr, bgr, lggr, lgbr, lngr, lnbr):
    tok = lambda i: (i, 0)
    cst = lambda i: (0, 0)
    return pl.pallas_call(
        _assemble_body,
        grid=(NBLK3,),
        in_specs=[
            pl.BlockSpec((TB3, 2 * D), tok),
            pl.BlockSpec((TB3, 2 * D), tok),
            pl.BlockSpec((TB3, D), tok),
            pl.BlockSpec((TB3, D), tok),
            pl.BlockSpec((2 * D, 2), cst),
            pl.BlockSpec((1, D), cst),
            pl.BlockSpec((1, 2), cst),
            pl.BlockSpec((1, 2), cst),
            pl.BlockSpec((1, 2), cst),
            pl.BlockSpec((1, D), cst),
            pl.BlockSpec((1, D), cst),
        ],
        out_specs=[
            pl.BlockSpec((TB3, D), tok),
            pl.BlockSpec((TB3, D), tok),
        ],
        out_shape=[
            jax.ShapeDtypeStruct((N, D), jnp.float32),
            jax.ShapeDtypeStruct((N, D), jnp.float32),
        ],
    )(qrows, nrows, xq, pos2d, wgt, bpr, bgr, lggr, lgbr, lngr, lnbr)


def _pos_encoding(b, l, d):
    h = w = int(math.sqrt(l))
    mask = jnp.ones((b, h, w), dtype=jnp.float32)
    y_embed = jnp.cumsum(mask, axis=1)
    x_embed = jnp.cumsum(mask, axis=2)
    eps = 1e-6
    y_embed = y_embed / (y_embed[:, -1:, :] + eps) * 2 * math.pi
    x_embed = x_embed / (x_embed[:, :, -1:] + eps) * 2 * math.pi
    pfd = d // 2
    dim_t = jnp.arange(pfd, dtype=jnp.float32)
    dim_t = 10000.0 ** (2.0 * jnp.floor(dim_t / 2.0) / pfd)
    pos_x = x_embed[:, :, :, None] / dim_t
    pos_y = y_embed[:, :, :, None] / dim_t
    pos_x = jnp.stack((jnp.sin(pos_x[:, :, :, 0::2]),
                       jnp.cos(pos_x[:, :, :, 1::2])), axis=4).reshape(b, h, w, pfd)
    pos_y = jnp.stack((jnp.sin(pos_y[:, :, :, 0::2]),
                       jnp.cos(pos_y[:, :, :, 1::2])), axis=4).reshape(b, h, w, pfd)
    pos = jnp.concatenate((pos_y, pos_x), axis=3)
    return pos.reshape(b, h * w, 2 * pfd)


def _np_threefry2x32(k0, k1, x0, x1):
    import numpy as np
    rot1 = (13, 15, 26, 6)
    rot2 = (17, 29, 16, 24)
    ks0, ks1 = np.uint32(k0), np.uint32(k1)
    ks2 = np.uint32(0x1BD11BDA) ^ ks0 ^ ks1
    x0 = (x0 + ks0).astype(np.uint32)
    x1 = (x1 + ks1).astype(np.uint32)
    add_idx = ((ks1, ks2), (ks2, ks0), (ks0, ks1), (ks1, ks2), (ks2, ks0))
    for r in range(5):
        for rot in (rot1 if r % 2 == 0 else rot2):
            x0 = (x0 + x1).astype(np.uint32)
            x1 = ((x1 << np.uint32(rot)) | (x1 >> np.uint32(32 - rot))).astype(np.uint32)
            x1 = x1 ^ x0
        ka, kb = add_idx[r]
        x0 = (x0 + ka).astype(np.uint32)
        x1 = (x1 + kb + np.uint32(r + 1)).astype(np.uint32)
    return x0, x1


def _np_gumbel(seed_pair, shape):
    import numpy as np
    n = int(np.prod(shape))
    o0, o1 = _np_threefry2x32(seed_pair[0], seed_pair[1],
                              np.zeros(n, np.uint32), np.arange(n, dtype=np.uint32))
    bits = o0 ^ o1
    f = ((bits >> np.uint32(9)) | np.uint32(0x3F800000)).view(np.float32)
    u = f - np.float32(1.0)
    tiny = np.float32(np.finfo(np.float32).tiny)
    u = np.maximum(tiny, (u * (np.float32(1.0) - tiny) + tiny).astype(np.float32))
    return (-np.log(-np.log(u))).astype(np.float32).reshape(shape)


def _np_pos_encoding(b, l, d):
    import numpy as np
    h = w = int(math.sqrt(l))
    f32 = np.float32
    y = np.cumsum(np.ones((b, h, w), f32), axis=1, dtype=f32)
    x = np.cumsum(np.ones((b, h, w), f32), axis=2, dtype=f32)
    eps = f32(1e-6)
    two_pi = f32(2 * math.pi)
    y = (y / (y[:, -1:, :] + eps) * two_pi).astype(f32)
    x = (x / (x[:, :, -1:] + eps) * two_pi).astype(f32)
    pfd = d // 2
    dim_t = np.arange(pfd, dtype=f32)
    dim_t = np.power(f32(10000.0), (f32(2.0) * np.floor(dim_t / f32(2.0)) / f32(pfd))).astype(f32)
    pos_x = (x[:, :, :, None] / dim_t).astype(f32)
    pos_y = (y[:, :, :, None] / dim_t).astype(f32)
    pos_x = np.stack((np.sin(pos_x[:, :, :, 0::2]), np.cos(pos_x[:, :, :, 1::2])),
                     axis=4).astype(f32).reshape(b, h, w, pfd)
    pos_y = np.stack((np.sin(pos_y[:, :, :, 0::2]), np.cos(pos_y[:, :, :, 1::2])),
                     axis=4).astype(f32).reshape(b, h, w, pfd)
    return np.concatenate((pos_y, pos_x), axis=3).reshape(b, h * w, 2 * pfd)


def _host_constants():
    import numpy as np
    g = _np_gumbel((0, 42), (N, TOPK))
    gpad = np.concatenate([g, np.zeros((N, 8 - TOPK), np.float32)], axis=1)
    pos = _np_pos_encoding(B, L, D).reshape(N, D)
    return gpad, pos


_GPAD_CONST, _POS_CONST = _host_constants()


def kernel(img, mask_indices, W1, b1, ln1_g, ln1_b, embed, Wp, bp, Wg, bg,
           lng_g, lng_b, ln_g, ln_b):
    img2d = img.reshape(N, DIN)
    w1t = W1.T
    embt = embed.T * 2.0
    b1r = b1.reshape(1, D)
    g1r = ln1_g.reshape(1, D)
    be1r = ln1_b.reshape(1, D)

    esqr = jnp.sum(embed ** 2, axis=1).reshape(1, K)
    xq, idxblk = _call_k1(img2d, w1t, b1r, g1r, be1r, embt, esqr)
    indices = idxblk.reshape(N)
    idx2d = indices.reshape(B, L)

    gpad = jnp.asarray(_GPAD_CONST)
    mask_pos = mask_indices.astype(jnp.int32).reshape(B, 1)

    neg2d = _call_k2(mask_pos, xq, embt, esqr, idx2d, gpad)

    table = _call_k0(embed, Wp.T)
    qrows, nrows = _sc_gather_two(table, indices, neg2d.reshape(N))

    pos2d = jnp.asarray(_POS_CONST)
    out1, out2 = _call_k3(
        qrows, nrows, xq, pos2d,
        Wg.T, bp.reshape(1, D), bg.reshape(1, 2),
        lng_g.reshape(1, 2), lng_b.reshape(1, 2),
        ln_g.reshape(1, D), ln_b.reshape(1, D),
    )
    return out1.reshape(B, L, D), out2.reshape(B, L, D)


# speedup vs baseline: 1.0046x; 1.0001x over previous
import functools
import math

import jax
import jax.numpy as jnp
from jax import lax
from jax.experimental import pallas as pl
from jax.experimental.pallas import tpu as pltpu
from jax.experimental.pallas import tpu_sc as plsc

B, L, DIN, D, K, TOPK = 8, 1024, 768, 64, 8192, 5
N = B * L
TB = 512
TB3 = 512
KC = 2048
NBLK = N // TB
NBLK3 = N // TB3


def _xq_argmin_body(img_ref, w1t_ref, b1_ref, g1_ref, be1_ref, embt_ref,
                    esq_ref, xq_ref, idx_ref):
    x = img_ref[...]
    h = jnp.dot(x, w1t_ref[...], preferred_element_type=jnp.float32)
    h = h + b1_ref[...]
    m = jnp.mean(h, axis=1, keepdims=True)
    c = h - m
    v = jnp.mean(c * c, axis=1, keepdims=True)
    xq = c / jnp.sqrt(v + 1e-5) * g1_ref[...] + be1_ref[...]
    xq = jnp.maximum(xq, 0.0)

    fsq = jnp.sum(xq * xq, axis=1, keepdims=True)
    rmin = None
    ridx = None
    for ci in range(K // KC):
        et = embt_ref[:, ci * KC:(ci + 1) * KC]
        esq = esq_ref[:, ci * KC:(ci + 1) * KC]
        mm2 = jnp.dot(xq, et, preferred_element_type=jnp.float32)
        d2 = (fsq + esq) - mm2
        cmin = jnp.min(d2, axis=1, keepdims=True)
        io = lax.broadcasted_iota(jnp.int32, (TB, KC), 1)
        cidx = jnp.min(jnp.where(d2 <= cmin, io, jnp.int32(1 << 30)),
                       axis=1, keepdims=True) + ci * KC
        if ci == 0:
            rmin, ridx = cmin, cidx
        else:
            upd = cmin < rmin
            rmin = jnp.where(upd, cmin, rmin)
            ridx = jnp.where(upd, cidx, ridx)
    xq_ref[...] = xq
    ridx_f = ridx.astype(jnp.float32)
    idx_ref[...] = jnp.transpose(ridx_f, (1, 0)).reshape(1, 1, TB).astype(jnp.int32)


def _call_k1(img2d, w1t, b1r, g1r, be1r, embt, esqr):
    return pl.pallas_call(
        _xq_argmin_body,
        grid=(NBLK,),
        in_specs=[
            pl.BlockSpec((TB, DIN), lambda i: (i, 0)),
            pl.BlockSpec((DIN, D), lambda i: (0, 0)),
            pl.BlockSpec((1, D), lambda i: (0, 0)),
            pl.BlockSpec((1, D), lambda i: (0, 0)),
            pl.BlockSpec((1, D), lambda i: (0, 0)),
            pl.BlockSpec((D, K), lambda i: (0, 0)),
            pl.BlockSpec((1, K), lambda i: (0, 0)),
        ],
        out_specs=[
            pl.BlockSpec((TB, D), lambda i: (i, 0)),
            pl.BlockSpec((1, 1, TB), lambda i: (i, 0, 0)),
        ],
        out_shape=[
            jax.ShapeDtypeStruct((N, D), jnp.float32),
            jax.ShapeDtypeStruct((NBLK, 1, TB), jnp.int32),
        ],
    )(img2d, w1t, b1r, g1r, be1r, embt, esqr)


def _sampling_body(mp_ref, xq_ref, embt_ref, esq_ref, idx2_ref,
                   g_ref, neg_ref):
    rows = []
    grows = []
    mposv = []
    for b in range(B):
        mpos = mp_ref[b, 0]
        t = b * L + mpos
        rows.append(xq_ref[pl.ds(t, 1), :])
        grows.append(g_ref[pl.ds(t, 1), :])
        mposv.append(mpos)
    rows = jnp.concatenate(rows, axis=0)
    grows = jnp.concatenate(grows, axis=0)

    ind2d = idx2_ref[...]
    iol = lax.broadcasted_iota(jnp.int32, (B, L), 1)
    mcol = jnp.concatenate([jnp.full((1, L), m, jnp.int32) for m in mposv], 0)
    oh = (iol == mcol).astype(jnp.float32)
    cb = jnp.sum(oh * ind2d.astype(jnp.float32), axis=1,
                 keepdims=True).astype(jnp.int32)

    et = embt_ref[...]
    esq = esq_ref[...]
    fsq = jnp.sum(rows * rows, axis=1, keepdims=True)
    mm2 = jnp.dot(rows, et, preferred_element_type=jnp.float32)
    d2 = (fsq + esq) - mm2
    nl = -d2
    mx = jnp.max(nl, axis=1, keepdims=True)
    p = jnp.exp(nl - mx)
    probs = p / jnp.sum(p, axis=1, keepdims=True)

    io = lax.broadcasted_iota(jnp.int32, (B, K), 1)
    pw = probs
    vals = []
    idxs = []
    for _ in range(TOPK):
        vmax = jnp.max(pw, axis=1, keepdims=True)
        ik = jnp.min(jnp.where(pw >= vmax, io, jnp.int32(1 << 30)),
                     axis=1, keepdims=True)
        vals.append(vmax)
        idxs.append(ik)
        pw = jnp.where(io == ik, -1.0, pw)

    best_l = jnp.log(vals[0] + 1e-12) + grows[:, 0:1]
    best_k = jnp.zeros((B, 1), jnp.int32)
    for k in range(1, TOPK):
        lk = jnp.log(vals[k] + 1e-12) + grows[:, k:k + 1]
        upd = lk > best_l
        best_l = jnp.where(upd, lk, best_l)
        best_k = jnp.where(upd, jnp.int32(k), best_k)
    ng = jnp.zeros((B, 1), jnp.int32)
    for k in range(TOPK):
        ng = jnp.where(best_k == k, idxs[k], ng)

    neg_ref[...] = jnp.where(ind2d == cb, ng, ind2d)


def _call_k2(mask_pos, xq, embt, esqr, idx2d, gpad):
    return pl.pallas_call(
        _sampling_body,
        in_specs=[
            pl.BlockSpec(memory_space=pltpu.SMEM),
            pl.BlockSpec((N, D), lambda: (0, 0)),
            pl.BlockSpec((D, K), lambda: (0, 0)),
            pl.BlockSpec((1, K), lambda: (0, 0)),
            pl.BlockSpec((B, L), lambda: (0, 0)),
            pl.BlockSpec((N, 8), lambda: (0, 0)),
        ],
        out_specs=pl.BlockSpec((B, L), lambda: (0, 0)),
        out_shape=jax.ShapeDtypeStruct((B, L), jnp.int32),
    )(mask_pos, xq, embt, esqr, idx2d, gpad)


def _table_body(emb_ref, wpt_ref, out_ref):
    e = emb_ref[...]
    out_ref[:, :D] = e.astype(jnp.bfloat16).astype(jnp.float32)
    out_ref[:, D:] = jnp.dot(e, wpt_ref[...],
                             preferred_element_type=jnp.float32)


def _call_k0(embed, wpt):
    return pl.pallas_call(
        _table_body,
        in_specs=[
            pl.BlockSpec((K, D), lambda: (0, 0)),
            pl.BlockSpec((D, D), lambda: (0, 0)),
        ],
        out_specs=pl.BlockSpec((K, 2 * D), lambda: (0, 0)),
        out_shape=jax.ShapeDtypeStruct((K, 2 * D), jnp.float32),
    )(embed, wpt)


_SC_ROWS = 256
_SC_CHUNK = 256


def _sc_gather_two(table, idx_a, idx_b):
    info = plsc.get_sparse_core_info()
    nw = info.num_cores * info.num_subcores
    mesh = plsc.VectorSubcoreMesh(core_axis_name="c", subcore_axis_name="s")

    nj = _SC_ROWS // _SC_CHUNK

    @functools.partial(
        pl.kernel,
        mesh=mesh,
        out_type=[
            jax.ShapeDtypeStruct((N, 2 * D), jnp.float32),
            jax.ShapeDtypeStruct((N, 2 * D), jnp.float32),
        ],
        scratch_types=(
            [pltpu.VMEM((_SC_CHUNK,), jnp.int32) for _ in range(2 * nj)]
            + [pltpu.VMEM((_SC_CHUNK, 2 * D), jnp.float32) for _ in range(2 * nj)]
            + [pltpu.SemaphoreType.DMA] * 3 * 2 * (_SC_ROWS // _SC_CHUNK)
        ),
    )
    def gather_k(table_hbm, ia_hbm, ib_hbm, out_a, out_b, *scratch):
        idx_vs = scratch[:2 * nj]
        row_vs = scratch[2 * nj:4 * nj]
        sems = scratch[4 * nj:]
        sem_i = sems[0:2 * nj]
        sem_g = sems[2 * nj:4 * nj]
        sem_s = sems[4 * nj:6 * nj]
        wid = lax.axis_index("s") * info.num_cores + lax.axis_index("c")
        plan = []
        k = 0
        for src, dst in ((ia_hbm, out_a), (ib_hbm, out_b)):
            for j in range(nj):
                base = wid * _SC_ROWS + j * _SC_CHUNK
                plan.append((src, dst, base, idx_vs[k], row_vs[k]))
                k += 1
        d = [pltpu.async_copy(src.at[pl.ds(base, _SC_CHUNK)], iv, sem_i[t])
             for t, (src, _, base, iv, _) in enumerate(plan)]
        g = []
        for t, (_, _, _, iv, rv) in enumerate(plan):
            d[t].wait()
            g.append(pltpu.async_copy(table_hbm.at[iv], rv, sem_g[t]))
        st = []
        for t, (_, dst, base, _, rv) in enumerate(plan):
            g[t].wait()
            st.append(pltpu.async_copy(rv, dst.at[pl.ds(base, _SC_CHUNK)],
                                       sem_s[t]))
        for x in st:
            x.wait()

    assert N % (8 * nw) == 0
    return gather_k(table, idx_a, idx_b)


def _assemble_body(q_ref, n_ref, xq_ref, pos_ref, wgt_ref, bp_ref, bg_ref,
                   lgg_ref, lgb_ref, lng_ref, lnb_ref, o1_ref, o2_ref):
    x = xq_ref[...]
    e = q_ref[:, D:] + bp_ref[...]
    nq = n_ref[:, :D]
    pos = pos_ref[...]

    tmp = jnp.concatenate([e, x], axis=1)
    sg = jnp.dot(tmp, wgt_ref[...],
                 preferred_element_type=jnp.float32) + bg_ref[...]
    s0 = sg[:, 0:1]
    s1 = sg[:, 1:2]
    mu = (s0 + s1) * 0.5
    d0 = s0 - mu
    d1 = s1 - mu
    var = (d0 * d0 + d1 * d1) * 0.5
    rs = jnp.sqrt(var + 1e-5)
    r0 = jnp.maximum(d0 / rs * lgg_ref[:, 0:1] + lgb_ref[:, 0:1], 0.0)
    r1 = jnp.maximum(d1 / rs * lgg_ref[:, 1:2] + lgb_ref[:, 1:2], 0.0)
    smx = jnp.maximum(r0, r1)
    e0 = jnp.exp(r0 - smx)
    e1 = jnp.exp(r1 - smx)
    den = e0 + e1
    es = e0 / den
    iss = e1 / den

    def final_ln(o):
        m = jnp.mean(o, axis=1, keepdims=True)
        c = o - m
        v = jnp.mean(c * c, axis=1, keepdims=True)
        return c / jnp.sqrt(v + 1e-5) * lng_ref[...] + lnb_ref[...]

    o1_ref[...] = final_ln(e * es + x * iss + pos)
    o2_ref[...] = final_ln(nq * es + x * iss + pos)


def _call_k3(qrows, nrows, xq, pos2d, wgt, bpr, bgr, lggr, lgbr, lngr, lnbr):
    tok = lambda i: (i, 0)
    cst = lambda i: (0, 0)
    return pl.pallas_call(
        _assemble_body,
        grid=(NBLK3,),
        in_specs=[
            pl.BlockSpec((TB3, 2 * D), tok),
            pl.BlockSpec((TB3, 2 * D), tok),
            pl.BlockSpec((TB3, D), tok),
            pl.BlockSpec((TB3, D), tok),
            pl.BlockSpec((2 * D, 2), cst),
            pl.BlockSpec((1, D), cst),
            pl.BlockSpec((1, 2), cst),
            pl.BlockSpec((1, 2), cst),
            pl.BlockSpec((1, 2), cst),
            pl.BlockSpec((1, D), cst),
            pl.BlockSpec((1, D), cst),
        ],
        out_specs=[
            pl.BlockSpec((TB3, D), tok),
            pl.BlockSpec((TB3, D), tok),
        ],
        out_shape=[
            jax.ShapeDtypeStruct((N, D), jnp.float32),
            jax.ShapeDtypeStruct((N, D), jnp.float32),
        ],
    )(qrows, nrows, xq, pos2d, wgt, bpr, bgr, lggr, lgbr, lngr, lnbr)


def _np_threefry2x32(k0, k1, x0, x1):
    import numpy as np
    rot1 = (13, 15, 26, 6)
    rot2 = (17, 29, 16, 24)
    ks0, ks1 = np.uint32(k0), np.uint32(k1)
    ks2 = np.uint32(0x1BD11BDA) ^ ks0 ^ ks1
    x0 = (x0 + ks0).astype(np.uint32)
    x1 = (x1 + ks1).astype(np.uint32)
    add_idx = ((ks1, ks2), (ks2, ks0), (ks0, ks1), (ks1, ks2), (ks2, ks0))
    for r in range(5):
        for rot in (rot1 if r % 2 == 0 else rot2):
            x0 = (x0 + x1).astype(np.uint32)
            x1 = ((x1 << np.uint32(rot)) | (x1 >> np.uint32(32 - rot))).astype(np.uint32)
            x1 = x1 ^ x0
        ka, kb = add_idx[r]
        x0 = (x0 + ka).astype(np.uint32)
        x1 = (x1 + kb + np.uint32(r + 1)).astype(np.uint32)
    return x0, x1


def _np_gumbel(seed_pair, shape):
    import numpy as np
    n = int(np.prod(shape))
    o0, o1 = _np_threefry2x32(seed_pair[0], seed_pair[1],
                              np.zeros(n, np.uint32), np.arange(n, dtype=np.uint32))
    bits = o0 ^ o1
    f = ((bits >> np.uint32(9)) | np.uint32(0x3F800000)).view(np.float32)
    u = f - np.float32(1.0)
    tiny = np.float32(np.finfo(np.float32).tiny)
    u = np.maximum(tiny, (u * (np.float32(1.0) - tiny) + tiny).astype(np.float32))
    return (-np.log(-np.log(u))).astype(np.float32).reshape(shape)


def _np_pos_encoding(b, l, d):
    import numpy as np
    h = w = int(math.sqrt(l))
    f32 = np.float32
    y = np.cumsum(np.ones((b, h, w), f32), axis=1, dtype=f32)
    x = np.cumsum(np.ones((b, h, w), f32), axis=2, dtype=f32)
    eps = f32(1e-6)
    two_pi = f32(2 * math.pi)
    y = (y / (y[:, -1:, :] + eps) * two_pi).astype(f32)
    x = (x / (x[:, :, -1:] + eps) * two_pi).astype(f32)
    pfd = d // 2
    dim_t = np.arange(pfd, dtype=f32)
    dim_t = np.power(f32(10000.0), (f32(2.0) * np.floor(dim_t / f32(2.0)) / f32(pfd))).astype(f32)
    pos_x = (x[:, :, :, None] / dim_t).astype(f32)
    pos_y = (y[:, :, :, None] / dim_t).astype(f32)
    pos_x = np.stack((np.sin(pos_x[:, :, :, 0::2]), np.cos(pos_x[:, :, :, 1::2])),
                     axis=4).astype(f32).reshape(b, h, w, pfd)
    pos_y = np.stack((np.sin(pos_y[:, :, :, 0::2]), np.cos(pos_y[:, :, :, 1::2])),
                     axis=4).astype(f32).reshape(b, h, w, pfd)
    return np.concatenate((pos_y, pos_x), axis=3).reshape(b, h * w, 2 * pfd)


def _host_constants():
    import numpy as np
    g = _np_gumbel((0, 42), (N, TOPK))
    gpad = np.concatenate([g, np.zeros((N, 8 - TOPK), np.float32)], axis=1)
    pos = _np_pos_encoding(B, L, D).reshape(N, D)
    return gpad, pos


_GPAD_CONST, _POS_CONST = _host_constants()


def kernel(img, mask_indices, W1, b1, ln1_g, ln1_b, embed, Wp, bp, Wg, bg,
           lng_g, lng_b, ln_g, ln_b):
    img2d = img.reshape(N, DIN)
    w1t = W1.T
    embt = embed.T * 2.0
    b1r = b1.reshape(1, D)
    g1r = ln1_g.reshape(1, D)
    be1r = ln1_b.reshape(1, D)

    esqr = jnp.sum(embed ** 2, axis=1).reshape(1, K)
    xq, idxblk = _call_k1(img2d, w1t, b1r, g1r, be1r, embt, esqr)
    indices = idxblk.reshape(N)
    idx2d = indices.reshape(B, L)

    gpad = jnp.asarray(_GPAD_CONST)
    mask_pos = mask_indices.astype(jnp.int32).reshape(B, 1)

    neg2d = _call_k2(mask_pos, xq, embt, esqr, idx2d, gpad)

    table = _call_k0(embed, Wp.T)
    qrows, nrows = _sc_gather_two(table, indices, neg2d.reshape(N))

    pos2d = jnp.asarray(_POS_CONST)
    out1, out2 = _call_k3(
        qrows, nrows, xq, pos2d,
        Wg.T, bp.reshape(1, D), bg.reshape(1, 2),
        lng_g.reshape(1, 2), lng_b.reshape(1, 2),
        ln_g.reshape(1, D), ln_b.reshape(1, D),
    )
    return out1.reshape(B, L, D), out2.reshape(B, L, D)
